# baseline (speedup 1.0000x reference)
import jax
import jax.numpy as jnp
from jax import lax
from jax.experimental import pallas as pl
from jax.experimental.pallas import tpu as pltpu
from jax.experimental.pallas import tpu_sc as plsc

H = 16384
W = 200
K = 21
C = K * W + 2 * W
L = 16
NC, NS = 2, 16
NW = NC * NS
COLS_PER_W = H // NW
IB = 128
LG = IB // L
NBLK = COLS_PER_W // IB
CH = 8
NCH = W // CH
OH = K * CH
SR = OH + 2 * CH


def _sc_body(dist_hbm, angle_hbm, idx_hbm, ind_hbm, out_hbm,
             d_v, a_v, i_c0, n_c0, i_c1, n_c1,
             out0, out1, pos0, pos1,
             sem_da, sem_ic0, sem_ic1, sem_out0, sem_out1):
    wid = lax.axis_index("s") * NC + lax.axis_index("c")
    i_base_w = wid * COLS_PER_W

    lane = lax.iota(jnp.int32, L)
    glane = [g * L + lane for g in range(LG)]
    ones = jnp.full((L,), 1.0, jnp.float32)
    zeros = jnp.zeros((L,), jnp.float32)

    outb = ((out0, pos0, sem_out0), (out1, pos1, sem_out1))
    inb = ((i_c0, n_c0, sem_ic0), (i_c1, n_c1, sem_ic1))

    def zero_body(r, c):
        for g in range(LG):
            out0[r, pl.ds(g * L, L)] = zeros
            out1[r, pl.ds(g * L, L)] = zeros
        return c
    lax.fori_loop(0, SR, zero_body, 0)

    def start_da(k):
        ib = i_base_w + k * IB
        pltpu.async_copy(dist_hbm.at[pl.ds(0, W), pl.ds(ib, IB)], d_v, sem_da)
        pltpu.async_copy(angle_hbm.at[pl.ds(0, W), pl.ds(ib, IB)], a_v, sem_da)

    def wait_da():
        pltpu.make_async_copy(dist_hbm.at[pl.ds(0, W), pl.ds(0, IB)], d_v,
                              sem_da).wait()
        pltpu.make_async_copy(angle_hbm.at[pl.ds(0, W), pl.ds(0, IB)], a_v,
                              sem_da).wait()

    def start_ic(k, c, Bi):
        ib = i_base_w + k * IB
        pltpu.async_copy(idx_hbm.at[pl.ds(c * CH, CH), pl.ds(ib, IB)],
                         Bi[0], Bi[2])
        pltpu.async_copy(ind_hbm.at[pl.ds(c * CH, CH), pl.ds(ib, IB)],
                         Bi[1], Bi[2])

    def wait_ic(Bi):
        pltpu.make_async_copy(idx_hbm.at[pl.ds(0, CH), pl.ds(0, IB)],
                              Bi[0], Bi[2]).wait()
        pltpu.make_async_copy(ind_hbm.at[pl.ds(0, CH), pl.ds(0, IB)],
                              Bi[1], Bi[2]).wait()

    def out_parts(k, c, Bo):
        ib = i_base_w + k * IB
        o_v = Bo[0]
        return (
            (o_v.at[pl.ds(0, OH), pl.ds(0, IB)],
             out_hbm.at[pl.ds(c * OH, OH), pl.ds(ib, IB)]),
            (o_v.at[pl.ds(OH, CH), pl.ds(0, IB)],
             out_hbm.at[pl.ds(K * W + c * CH, CH), pl.ds(ib, IB)]),
            (o_v.at[pl.ds(OH + CH, CH), pl.ds(0, IB)],
             out_hbm.at[pl.ds(K * W + W + c * CH, CH), pl.ds(ib, IB)]),
        )

    def start_out(k, c, Bo):
        for src, dst in out_parts(k, c, Bo):
            pltpu.async_copy(src, dst, Bo[2])

    def wait_out(Bo):
        for src, dst in out_parts(0, 0, Bo):
            pltpu.make_async_copy(src, dst, Bo[2]).wait()

    def compute_pass(Bo, Bi):
        o_v, p_v = Bo[0:2]
        i_c, n_c = Bi[0:2]
        for jj in range(CH):
            for g in range(LG):
                gl = glane[g]
                idxv = i_c[jj, pl.ds(g * L, L)]
                indv = n_c[jj, pl.ds(g * L, L)]
                cv = jj * K + idxv
                p_v[pl.ds((jj * LG + g) * L, L)] = cv
                plsc.store_scatter(o_v, [cv, gl], ones)
                dd = plsc.load_gather(d_v, [indv, gl]) * jnp.float32(0.1)
                aa = plsc.load_gather(a_v, [indv, gl]) * jnp.float32(1.0 / 3.0)
                o_v[OH + jj, pl.ds(g * L, L)] = dd
                o_v[OH + CH + jj, pl.ds(g * L, L)] = aa

    def rezero_pass(Bo):
        o_v, p_v = Bo[0:2]
        for jj in range(CH):
            for g in range(LG):
                cv = p_v[pl.ds((jj * LG + g) * L, L)]
                plsc.store_scatter(o_v, [cv, glane[g]], zeros)

    def blk(k, carry):
        start_da(k)
        start_ic(k, 0, inb[0])
        start_ic(k, 1, inb[1])
        wait_da()

        def chunk(c, cc):
            p = k * NCH + c
            q = lax.rem(c, 2)
            for P in range(2):
                @pl.when(q == P)
                def _():
                    Bo = outb[P]
                    Bi = inb[P]
                    wait_ic(Bi)

                    @pl.when(p >= 2)
                    def _():
                        wait_out(Bo)
                        rezero_pass(Bo)

                    compute_pass(Bo, Bi)
                    start_out(k, c, Bo)

                    @pl.when(c + 2 < NCH)
                    def _():
                        start_ic(k, c + 2, Bi)
            return cc
        lax.fori_loop(0, NCH, chunk, 0)
        return carry

    lax.fori_loop(0, NBLK, blk, 0)
    wait_out(outb[0])
    wait_out(outb[1])


_sc_call = pl.kernel(
    _sc_body,
    out_type=jax.ShapeDtypeStruct((C, H), jnp.float32),
    mesh=plsc.VectorSubcoreMesh(core_axis_name="c", subcore_axis_name="s",
                                num_cores=NC, num_subcores=NS),
    scratch_types=[
        pltpu.VMEM((W, IB), jnp.float32),
        pltpu.VMEM((W, IB), jnp.float32),
        pltpu.VMEM((CH, IB), jnp.int32),
        pltpu.VMEM((CH, IB), jnp.int32),
        pltpu.VMEM((CH, IB), jnp.int32),
        pltpu.VMEM((CH, IB), jnp.int32),
        pltpu.VMEM((SR, IB), jnp.float32),
        pltpu.VMEM((SR, IB), jnp.float32),
        pltpu.VMEM((CH * LG * L,), jnp.int32),
        pltpu.VMEM((CH * LG * L,), jnp.int32),
        pltpu.SemaphoreType.DMA,
        pltpu.SemaphoreType.DMA,
        pltpu.SemaphoreType.DMA,
        pltpu.SemaphoreType.DMA,
        pltpu.SemaphoreType.DMA,
    ],
    compiler_params=pltpu.CompilerParams(needs_layout_passes=False,
                                         skip_device_barrier=True),
)


@jax.jit
def kernel(dist, angle, idx_t, index_t):
    out_t = _sc_call(dist.T, angle.T,
                     idx_t.astype(jnp.int32).T,
                     index_t.astype(jnp.int32).T)
    return out_t.T


# speedup vs baseline: 1.0001x; 1.0001x over previous
import jax
import jax.numpy as jnp
from jax import lax
from jax.experimental import pallas as pl
from jax.experimental.pallas import tpu as pltpu
from jax.experimental.pallas import tpu_sc as plsc

H = 16384
W = 200
K = 21
C = K * W + 2 * W
L = 16
NC, NS = 2, 16
NW = NC * NS
COLS_PER_W = H // NW
IB = 128
LG = IB // L
NBLK = COLS_PER_W // IB
CH = 8
NCH = W // CH
OH = K * CH
SR = OH + 2 * CH


def _sc_body(dist_hbm, angle_hbm, idx_hbm, ind_hbm, out_hbm,
             d_v, a_v, i_c0, n_c0, i_c1, n_c1,
             out0, out1, pos0, pos1,
             sem_da, sem_ic0, sem_ic1, sem_out0, sem_out1):
    wid = lax.axis_index("s") * NC + lax.axis_index("c")
    i_base_w = wid * COLS_PER_W

    lane = lax.iota(jnp.int32, L)
    glane = [g * L + lane for g in range(LG)]
    ones = jnp.full((L,), 1.0, jnp.float32)
    zeros = jnp.zeros((L,), jnp.float32)

    outb = ((out0, pos0, sem_out0), (out1, pos1, sem_out1))
    inb = ((i_c0, n_c0, sem_ic0), (i_c1, n_c1, sem_ic1))

    def zero_body(r, c):
        for g in range(LG):
            out0[r, pl.ds(g * L, L)] = zeros
            out1[r, pl.ds(g * L, L)] = zeros
        return c
    lax.fori_loop(0, SR, zero_body, 0)

    def start_da(k):
        ib = i_base_w + k * IB
        pltpu.async_copy(dist_hbm.at[pl.ds(0, W), pl.ds(ib, IB)], d_v, sem_da)
        pltpu.async_copy(angle_hbm.at[pl.ds(0, W), pl.ds(ib, IB)], a_v, sem_da)

    def wait_da():
        pltpu.make_async_copy(dist_hbm.at[pl.ds(0, W), pl.ds(0, IB)], d_v,
                              sem_da).wait()
        pltpu.make_async_copy(angle_hbm.at[pl.ds(0, W), pl.ds(0, IB)], a_v,
                              sem_da).wait()

    def start_ic(k, c, Bi):
        ib = i_base_w + k * IB
        pltpu.async_copy(idx_hbm.at[pl.ds(c * CH, CH), pl.ds(ib, IB)],
                         Bi[0], Bi[2])
        pltpu.async_copy(ind_hbm.at[pl.ds(c * CH, CH), pl.ds(ib, IB)],
                         Bi[1], Bi[2])

    def wait_ic(Bi):
        pltpu.make_async_copy(idx_hbm.at[pl.ds(0, CH), pl.ds(0, IB)],
                              Bi[0], Bi[2]).wait()
        pltpu.make_async_copy(ind_hbm.at[pl.ds(0, CH), pl.ds(0, IB)],
                              Bi[1], Bi[2]).wait()

    def out_parts(k, c, Bo):
        ib = i_base_w + k * IB
        o_v = Bo[0]
        return (
            (o_v.at[pl.ds(0, OH), pl.ds(0, IB)],
             out_hbm.at[pl.ds(c * OH, OH), pl.ds(ib, IB)]),
            (o_v.at[pl.ds(OH, CH), pl.ds(0, IB)],
             out_hbm.at[pl.ds(K * W + c * CH, CH), pl.ds(ib, IB)]),
            (o_v.at[pl.ds(OH + CH, CH), pl.ds(0, IB)],
             out_hbm.at[pl.ds(K * W + W + c * CH, CH), pl.ds(ib, IB)]),
        )

    def start_out(k, c, Bo):
        for src, dst in out_parts(k, c, Bo):
            pltpu.async_copy(src, dst, Bo[2])

    def wait_out(Bo):
        for src, dst in out_parts(0, 0, Bo):
            pltpu.make_async_copy(src, dst, Bo[2]).wait()

    def compute_pass(Bo, Bi):
        o_v, p_v = Bo[0:2]
        i_c, n_c = Bi[0:2]
        for jj in range(CH):
            for g in range(LG):
                gl = glane[g]
                idxv = i_c[jj, pl.ds(g * L, L)]
                indv = n_c[jj, pl.ds(g * L, L)]
                cv = jj * K + idxv
                p_v[pl.ds((jj * LG + g) * L, L)] = cv
                plsc.store_scatter(o_v, [cv, gl], ones)
                dd = plsc.load_gather(d_v, [indv, gl]) * jnp.float32(0.1)
                aa = plsc.load_gather(a_v, [indv, gl]) * jnp.float32(1.0 / 3.0)
                o_v[OH + jj, pl.ds(g * L, L)] = dd
                o_v[OH + CH + jj, pl.ds(g * L, L)] = aa

    def rezero_pass(Bo):
        o_v, p_v = Bo[0:2]
        for jj in range(CH):
            for g in range(LG):
                cv = p_v[pl.ds((jj * LG + g) * L, L)]
                plsc.store_scatter(o_v, [cv, glane[g]], zeros)

    def blk(k, carry):
        start_da(k)
        start_ic(k, 0, inb[0])
        start_ic(k, 1, inb[1])
        wait_da()

        def chunk(c, cc):
            p = k * NCH + c
            q = lax.rem(c, 2)
            for P in range(2):
                @pl.when(q == P)
                def _():
                    Bo = outb[P]
                    Bi = inb[P]
                    wait_ic(Bi)

                    @pl.when(p >= 2)
                    def _():
                        wait_out(Bo)
                        rezero_pass(Bo)

                    compute_pass(Bo, Bi)
                    start_out(k, c, Bo)

                    @pl.when(c + 2 < NCH)
                    def _():
                        start_ic(k, c + 2, Bi)
            return cc
        lax.fori_loop(0, NCH, chunk, 0)
        return carry

    lax.fori_loop(0, NBLK, blk, 0)
    wait_out(outb[0])
    wait_out(outb[1])


_sc_call = pl.kernel(
    _sc_body,
    out_type=jax.ShapeDtypeStruct((C, H), jnp.float32),
    mesh=plsc.VectorSubcoreMesh(core_axis_name="c", subcore_axis_name="s",
                                num_cores=NC, num_subcores=NS),
    scratch_types=[
        pltpu.VMEM((W, IB), jnp.float32),
        pltpu.VMEM((W, IB), jnp.float32),
        pltpu.VMEM((CH, IB), jnp.int32),
        pltpu.VMEM((CH, IB), jnp.int32),
        pltpu.VMEM((CH, IB), jnp.int32),
        pltpu.VMEM((CH, IB), jnp.int32),
        pltpu.VMEM((SR, IB), jnp.float32),
        pltpu.VMEM((SR, IB), jnp.float32),
        pltpu.VMEM((CH * LG * L,), jnp.int32),
        pltpu.VMEM((CH * LG * L,), jnp.int32),
        pltpu.SemaphoreType.DMA,
        pltpu.SemaphoreType.DMA,
        pltpu.SemaphoreType.DMA,
        pltpu.SemaphoreType.DMA,
        pltpu.SemaphoreType.DMA,
    ],
    compiler_params=pltpu.CompilerParams(needs_layout_passes=False),
)


@jax.jit
def kernel(dist, angle, idx_t, index_t):
    out_t = _sc_call(dist.T, angle.T,
                     idx_t.astype(jnp.int32).T,
                     index_t.astype(jnp.int32).T)
    return out_t.T
